# Initial kernel scaffold; baseline (speedup 1.0000x reference)
import jax
import jax.numpy as jnp
from jax.experimental import pallas as pl


def kernel(pred_points, true_points):
    raise NotImplementedError("write your pallas kernel here")


# speedup vs baseline: 1.7514x; 1.7514x over previous
import jax
import jax.numpy as jnp
from jax.experimental import pallas as pl

_NQ = 8192
_TQ = 128
_NT = _NQ // _TQ
_NB = 4
_ND = 2 * _NB


def _nn_body(q_ref, k_ref, min_ref, idx_ref, sum_ref):
    t = pl.program_id(1)
    q = q_ref[0]
    k = k_ref[0]
    dx = q[:, 0:1] - k[0:1, :]
    dy = q[:, 1:2] - k[1:2, :]
    dz = q[:, 2:3] - k[2:3, :]
    d = dx * dx + dy * dy + dz * dz
    m = jnp.min(d, axis=1)
    iota = jax.lax.broadcasted_iota(jnp.int32, d.shape, 1)
    hit = jnp.where(d == m[:, None], iota, jnp.int32(_NQ))
    idx = jnp.min(hit, axis=1)
    min_ref[0, 0, :] = m
    idx_ref[0, 0, :] = idx

    @pl.when(t == 0)
    def _():
        sum_ref[0, 0, :] = jnp.zeros((_TQ,), jnp.float32)

    sum_ref[0, 0, :] += m


def kernel(pred_points, true_points):
    qs = jnp.concatenate([pred_points, true_points], axis=0)
    ks = jnp.concatenate([true_points, pred_points], axis=0)
    ks = ks.transpose(0, 2, 1)

    grid = (_ND, _NT)
    mins, idxs, sums = pl.pallas_call(
        _nn_body,
        grid=grid,
        in_specs=[
            pl.BlockSpec((1, _TQ, 3), lambda b, t: (b, t, 0)),
            pl.BlockSpec((1, 3, _NQ), lambda b, t: (b, 0, 0)),
        ],
        out_specs=[
            pl.BlockSpec((1, 1, _TQ), lambda b, t: (b * _NT + t, 0, 0)),
            pl.BlockSpec((1, 1, _TQ), lambda b, t: (b * _NT + t, 0, 0)),
            pl.BlockSpec((1, 1, _TQ), lambda b, t: (b, 0, 0)),
        ],
        out_shape=[
            jax.ShapeDtypeStruct((_ND * _NT, 1, _TQ), jnp.float32),
            jax.ShapeDtypeStruct((_ND * _NT, 1, _TQ), jnp.int32),
            jax.ShapeDtypeStruct((_ND, 1, _TQ), jnp.float32),
        ],
    )(qs, ks)

    loss = jnp.sum(sums) / (_NB * _NQ)
    idxs = idxs.reshape(_ND, _NQ)
    return loss, idxs[:_NB], idxs[_NB:]
